# baseline (speedup 1.0000x reference)
import jax
import jax.numpy as jnp
from jax import lax
from jax.experimental import pallas as pl
from jax.experimental.pallas import tpu as pltpu
from jax.experimental.pallas import tpu_sc as plsc

EMBED = 32
LANES = 16
NUM_CORES = 2
NUM_SUBCORES = 16
NUM_WORKERS = NUM_CORES * NUM_SUBCORES
NBANDS = 4


def _mf_body(uid_hbm, mid_hbm, utabT_hbm, mtabT_hbm, out_hbm,
             ids_u, ids_m, bufs, du, dm, out_v, sems):
    b_per_w = ids_u.shape[0]
    wid = lax.axis_index("s") * NUM_CORES + lax.axis_index("c")
    base = wid * b_per_w

    pltpu.sync_copy(uid_hbm.at[pl.ds(base, b_per_w)], ids_u)
    pltpu.sync_copy(mid_hbm.at[pl.ds(base, b_per_w)], ids_m)

    jrow = lax.iota(jnp.int32, LANES)

    def stage_table(tab_hbm, ids_v, dst):
        rows8 = jrow & 7
        mask8 = jrow < 8

        def group(g, carry):
            vec = ids_v[pl.ds(g * LANES, LANES)]
            tiles = (vec >> 7) * 128
            lanes = vec & 127
            for j in range(LANES):
                t = pl.multiple_of(tiles[j], 128)
                for r in range(NBANDS):
                    pltpu.async_copy(
                        tab_hbm.at[pl.ds(r * 8, 8), pl.ds(t, 128)],
                        bufs.at[NBANDS * j + r], sems.at[j])
            pos = jrow + g * LANES
            for j in range(LANES):
                lane = jnp.full((LANES,), lanes[j], jnp.int32)
                cpos = jnp.full((LANES,), pos[j], jnp.int32)
                for r in range(NBANDS):
                    pltpu.make_async_copy(
                        tab_hbm.at[pl.ds(0, 8), pl.ds(0, 128)],
                        bufs.at[NBANDS * j + r], sems.at[j]).wait()
                for r in range(NBANDS):
                    slot = NBANDS * j + r
                    v = plsc.load_gather(bufs.at[slot], [rows8, lane])
                    plsc.store_scatter(dst, [rows8 + r * 8, cpos], v,
                                       mask=mask8)
            return carry

        lax.fori_loop(0, b_per_w // LANES, group, 0)

    stage_table(utabT_hbm, ids_u, du)
    stage_table(mtabT_hbm, ids_m, dm)

    def group_body(g, carry):
        cols = jrow + g * LANES
        acc = jnp.zeros((LANES,), jnp.float32)
        for e in range(EMBED):
            row = jnp.full((LANES,), e, jnp.int32)
            uv = plsc.load_gather(du, [row, cols])
            mv = plsc.load_gather(dm, [row, cols])
            acc = acc + uv * mv
        out_v[pl.ds(g * LANES, LANES)] = acc
        return carry

    lax.fori_loop(0, b_per_w // LANES, group_body, 0)

    pltpu.sync_copy(out_v, out_hbm.at[pl.ds(base, b_per_w)])


@jax.jit
def kernel(user_ids, movie_ids, user_table, movie_table):
    uid = user_ids.astype(jnp.int32)
    mid = movie_ids.astype(jnp.int32)
    batch = uid.shape[0]
    b_per_w = batch // NUM_WORKERS

    mesh = plsc.VectorSubcoreMesh(
        core_axis_name="c", subcore_axis_name="s",
        num_cores=NUM_CORES, num_subcores=NUM_SUBCORES)

    mf = pl.kernel(
        _mf_body,
        out_type=jax.ShapeDtypeStruct((batch,), jnp.float32),
        mesh=mesh,
        scratch_types=[
            pltpu.VMEM((b_per_w,), jnp.int32),
            pltpu.VMEM((b_per_w,), jnp.int32),
            pltpu.VMEM((NBANDS * LANES, 8, 128), jnp.float32),
            pltpu.VMEM((EMBED, b_per_w), jnp.float32),
            pltpu.VMEM((EMBED, b_per_w), jnp.float32),
            pltpu.VMEM((b_per_w,), jnp.float32),
            pltpu.SemaphoreType.DMA((LANES,)),
        ],
        compiler_params=pltpu.CompilerParams(needs_layout_passes=False),
    )
    return mf(uid, mid, user_table.T, movie_table.T)


# speedup vs baseline: 1.0291x; 1.0291x over previous
import jax
import jax.numpy as jnp
from jax import lax
from jax.experimental import pallas as pl
from jax.experimental.pallas import tpu as pltpu
from jax.experimental.pallas import tpu_sc as plsc

EMBED = 32
LANES = 16
NUM_CORES = 2
NUM_SUBCORES = 16
NUM_WORKERS = NUM_CORES * NUM_SUBCORES


def _mf_body(uid_hbm, mid_hbm, utabT_hbm, mtabT_hbm, out_hbm,
             ids_u, ids_m, bufs, du, dm, out_v, sems):
    b_per_w = ids_u.shape[0]
    wid = lax.axis_index("s") * NUM_CORES + lax.axis_index("c")
    base = wid * b_per_w

    pltpu.sync_copy(uid_hbm.at[pl.ds(base, b_per_w)], ids_u)
    pltpu.sync_copy(mid_hbm.at[pl.ds(base, b_per_w)], ids_m)

    jrow = lax.iota(jnp.int32, LANES)

    def stage_table(tab_hbm, ids_v, dst):
        def group(g, carry):
            vec = ids_v[pl.ds(g * LANES, LANES)]
            tiles = (vec >> 7) * 128
            lanes = vec & 127
            for j in range(LANES):
                t = pl.multiple_of(tiles[j], 128)
                pltpu.async_copy(tab_hbm.at[:, pl.ds(t, 128)],
                                 bufs.at[j], sems.at[j])
            pos = jrow + g * LANES
            for j in range(LANES):
                pltpu.make_async_copy(tab_hbm.at[:, pl.ds(0, 128)],
                                      bufs.at[j], sems.at[j]).wait()
                lane = jnp.full((LANES,), lanes[j], jnp.int32)
                cpos = jnp.full((LANES,), pos[j], jnp.int32)
                for h in range(EMBED // LANES):
                    rows = jrow + h * LANES
                    v = plsc.load_gather(bufs.at[j], [rows, lane])
                    plsc.store_scatter(dst, [rows, cpos], v)
            return carry

        lax.fori_loop(0, b_per_w // LANES, group, 0)

    stage_table(utabT_hbm, ids_u, du)
    stage_table(mtabT_hbm, ids_m, dm)

    def group_body(g, carry):
        cols = jrow + g * LANES
        acc = jnp.zeros((LANES,), jnp.float32)
        for e in range(EMBED):
            row = jnp.full((LANES,), e, jnp.int32)
            uv = plsc.load_gather(du, [row, cols])
            mv = plsc.load_gather(dm, [row, cols])
            acc = acc + uv * mv
        out_v[pl.ds(g * LANES, LANES)] = acc
        return carry

    lax.fori_loop(0, b_per_w // LANES, group_body, 0)

    pltpu.sync_copy(out_v, out_hbm.at[pl.ds(base, b_per_w)])


@jax.jit
def kernel(user_ids, movie_ids, user_table, movie_table):
    uid = user_ids.astype(jnp.int32)
    mid = movie_ids.astype(jnp.int32)
    batch = uid.shape[0]
    b_per_w = batch // NUM_WORKERS

    mesh = plsc.VectorSubcoreMesh(
        core_axis_name="c", subcore_axis_name="s",
        num_cores=NUM_CORES, num_subcores=NUM_SUBCORES)

    mf = pl.kernel(
        _mf_body,
        out_type=jax.ShapeDtypeStruct((batch,), jnp.float32),
        mesh=mesh,
        scratch_types=[
            pltpu.VMEM((b_per_w,), jnp.int32),
            pltpu.VMEM((b_per_w,), jnp.int32),
            pltpu.VMEM((LANES, EMBED, 128), jnp.float32),
            pltpu.VMEM((EMBED, b_per_w), jnp.float32),
            pltpu.VMEM((EMBED, b_per_w), jnp.float32),
            pltpu.VMEM((b_per_w,), jnp.float32),
            pltpu.SemaphoreType.DMA((LANES,)),
        ],
        compiler_params=pltpu.CompilerParams(needs_layout_passes=False),
    )
    return mf(uid, mid, user_table.T, movie_table.T)
